# Initial kernel scaffold; baseline (speedup 1.0000x reference)
import jax
import jax.numpy as jnp
from jax.experimental import pallas as pl


def kernel(x, W, b, noise):
    raise NotImplementedError("write your pallas kernel here")


# speedup vs baseline: 9.2235x; 9.2235x over previous
import jax
import jax.numpy as jnp
from jax.experimental import pallas as pl

_TOKENS = 8192
_D_MODEL = 2048
_NUM_EXPERTS = 64
_TOP_K = 8
_BLK_T = 512


def _router_block(x_ref, w_ref, b_ref, noise_t_ref, out_ref, idx_ref):
    x = x_ref[...]
    w = w_ref[...]
    l = jax.lax.dot_general(w, x, (((1,), (1,)), ((), ())),
                            preferred_element_type=jnp.float32)
    l = l + b_ref[...] + noise_t_ref[...]
    l0 = l
    iota_e = jax.lax.broadcasted_iota(jnp.int32, l.shape, 0)
    selected = jnp.zeros(l.shape, jnp.bool_)
    idx_rows = []
    m0 = None
    for k in range(_TOP_K):
        m = jnp.max(l, axis=0, keepdims=True)
        if k == 0:
            m0 = m
        hit = l == m
        idx = jnp.min(jnp.where(hit, iota_e, _NUM_EXPERTS), axis=0,
                      keepdims=True)
        sel = iota_e == idx
        selected = jnp.logical_or(selected, sel)
        l = jnp.where(sel, -jnp.inf, l)
        idx_rows.append(idx)
    probs = jnp.where(selected, jnp.exp(l0 - m0), 0.0)
    z = jnp.sum(probs, axis=0, keepdims=True)
    out_ref[...] = probs / z
    idx_ref[...] = jnp.concatenate(idx_rows, axis=0)


def kernel(x, W, b, noise):
    noise_t = noise.T
    b2 = b.reshape(_NUM_EXPERTS, 1)
    out_t, idx_t = pl.pallas_call(
        _router_block,
        grid=(_TOKENS // _BLK_T,),
        in_specs=[
            pl.BlockSpec((_BLK_T, _D_MODEL), lambda i: (i, 0)),
            pl.BlockSpec((_NUM_EXPERTS, _D_MODEL), lambda i: (0, 0)),
            pl.BlockSpec((_NUM_EXPERTS, 1), lambda i: (0, 0)),
            pl.BlockSpec((_NUM_EXPERTS, _BLK_T), lambda i: (0, i)),
        ],
        out_specs=[
            pl.BlockSpec((_NUM_EXPERTS, _BLK_T), lambda i: (0, i)),
            pl.BlockSpec((_TOP_K, _BLK_T), lambda i: (0, i)),
        ],
        out_shape=[
            jax.ShapeDtypeStruct((_NUM_EXPERTS, _TOKENS), jnp.float32),
            jax.ShapeDtypeStruct((_TOP_K, _TOKENS), jnp.int32),
        ],
    )(x, W, b2, noise_t)
    return out_t.T, idx_t.T
